# baseline (speedup 1.0000x reference)
import functools

import jax
import jax.numpy as jnp
from jax import lax
from jax.experimental import pallas as pl
from jax.experimental.pallas import tpu as pltpu
from jax.experimental.pallas import tpu_sc as plsc

E = 64
HIDDEN = 1024
FFN = 2048
T = 8192
TM = 256
NT = T // TM
NP = NT + E - 1


_NW = 32


def _make_row_gather(B, D, dtype, CH):
    bpw = B // _NW
    nch = bpw // CH
    mesh = plsc.VectorSubcoreMesh(core_axis_name="c", subcore_axis_name="s")

    @functools.partial(
        pl.kernel, mesh=mesh,
        out_type=jax.ShapeDtypeStruct((B, D), dtype),
        scratch_types=[
            pltpu.VMEM((nch, CH), jnp.int32),
            pltpu.VMEM((2, CH, D), dtype),
            pltpu.SemaphoreType.DMA,
            pltpu.SemaphoreType.DMA,
        ],
    )
    def gather_k(table_hbm, idx_hbm, out_hbm, idx_v, rows_v, sem0, sem1):
        wid = lax.axis_index("s") * 2 + lax.axis_index("c")
        base = wid * bpw
        pltpu.sync_copy(idx_hbm.at[pl.ds(wid * nch, nch)], idx_v)
        sems = [sem0, sem1]
        cps = [None, None]
        for c in range(nch):
            b = c % 2
            cps[b] = pltpu.async_copy(
                table_hbm.at[idx_v.at[c]], rows_v.at[b], sems[b])
            if c > 0:
                pb = (c - 1) % 2
                cps[pb].wait()
                pltpu.sync_copy(rows_v.at[pb],
                                out_hbm.at[pl.ds(base + (c - 1) * CH, CH)])
        lb = (nch - 1) % 2
        cps[lb].wait()
        pltpu.sync_copy(rows_v.at[lb],
                        out_hbm.at[pl.ds(base + (nch - 1) * CH, CH)])

    return gather_k


_CHUNK = T // _NW
_XW = HIDDEN // 2
_SC_MESH = dict(core_axis_name="c", subcore_axis_name="s")


def _sc_wid():
    return lax.axis_index("s") * 2 + lax.axis_index("c")


@functools.partial(
    pl.kernel, mesh=plsc.VectorSubcoreMesh(**_SC_MESH),
    out_type=jax.ShapeDtypeStruct((_NW, E), jnp.int32),
    scratch_types=[
        pltpu.VMEM((_CHUNK,), jnp.int32),
        pltpu.VMEM((E,), jnp.int32),
        pltpu.SMEM((E,), jnp.int32),
    ],
)
def _sc_histogram(top_hbm, hist_hbm, keys_v, hist_v, cur_s):
    wid = _sc_wid()
    pltpu.sync_copy(top_hbm.at[pl.ds(wid * _CHUNK, _CHUNK)], keys_v)
    for e in range(E):
        cur_s[e] = 0
    for v in range(_CHUNK // 16):
        kvec = keys_v[pl.ds(v * 16, 16)]
        for l in range(16):
            e = kvec[l]
            cur_s[e] = cur_s[e] + 1
    lanes = lax.broadcasted_iota(jnp.int32, (16,), 0)
    for g in range(E // 16):
        vec = jnp.zeros((16,), jnp.int32)
        for l in range(16):
            vec = jnp.where(lanes == l, cur_s[g * 16 + l], vec)
        hist_v[pl.ds(g * 16, 16)] = vec
    pltpu.sync_copy(hist_v, hist_hbm.at[wid])


@functools.partial(
    pl.kernel, mesh=plsc.VectorSubcoreMesh(**_SC_MESH),
    out_type=(
        jax.ShapeDtypeStruct((T, _XW), jnp.int32),
        jax.ShapeDtypeStruct((T, 128), jnp.float32),
        jax.ShapeDtypeStruct((T // 128, 128), jnp.int32),
    ),
    scratch_types=[
        pltpu.VMEM((_CHUNK,), jnp.int32),
        pltpu.VMEM((2, 128), jnp.int32),
        pltpu.VMEM((128, _XW), jnp.int32),
        pltpu.VMEM((128, 128), jnp.float32),
        pltpu.VMEM((E,), jnp.int32),
        pltpu.SMEM((E,), jnp.int32),
        pltpu.SemaphoreType.DMA,
    ],
)
def _sc_place_scatter(top_hbm, start_hbm, hs_hbm, ew128_hbm,
                      xs_hbm, ews_hbm, inv_hbm,
                      keys_v, dest2_v, rows_v, ew_v, start_v, cur_s, sem):
    wid = _sc_wid()
    base = wid * _CHUNK
    pltpu.sync_copy(top_hbm.at[pl.ds(base, _CHUNK)], keys_v)
    pltpu.sync_copy(start_hbm.at[wid], start_v)
    for g in range(E // 16):
        svec = start_v[pl.ds(g * 16, 16)]
        for l in range(16):
            cur_s[g * 16 + l] = svec[l]
    lanes = lax.broadcasted_iota(jnp.int32, (16,), 0)
    for v in range(_CHUNK // 16):
        kvec = keys_v[pl.ds(v * 16, 16)]
        vec = jnp.zeros((16,), jnp.int32)
        for l in range(16):
            e = kvec[l]
            p = cur_s[e]
            cur_s[e] = p + 1
            vec = jnp.where(lanes == l, p, vec)
        dest2_v[v // 8, pl.ds((v % 8) * 16, 16)] = vec
    pltpu.sync_copy(dest2_v, inv_hbm.at[pl.ds(wid * 2, 2)])
    for c in range(2):
        pltpu.sync_copy(hs_hbm.at[pl.ds(base + c * 128, 128)], rows_v)
        pltpu.async_copy(rows_v, xs_hbm.at[dest2_v.at[c]], sem).wait()
        pltpu.sync_copy(ew128_hbm.at[pl.ds(base + c * 128, 128)], ew_v)
        pltpu.async_copy(ew_v, ews_hbm.at[dest2_v.at[c]], sem).wait()


def _moe_body(t_ids, g_ids, offs, nreal, x_ref, w1_ref, b1_ref, w2_ref,
              b2_ref, ew_ref, out_ref):
    i = pl.program_id(0)
    t = t_ids[i]
    g = g_ids[i]
    prev_t = t_ids[jnp.maximum(i - 1, 0)]
    first_visit = jnp.logical_or(i == 0, prev_t != t)

    @pl.when(first_visit)
    def _():
        out_ref[...] = jnp.zeros_like(out_ref)

    @pl.when(i < nreal[0])
    def _():
        row = t * TM + jax.lax.broadcasted_iota(jnp.int32, (TM, 1), 0)
        mask = jnp.logical_and(row >= offs[g], row < offs[g + 1])
        x = x_ref[...]
        fc1 = jnp.dot(x, w1_ref[0], preferred_element_type=jnp.float32)
        fc1 = fc1 + b1_ref[0]
        act = (0.5 * fc1 * (1.0 + jax.lax.erf(fc1 * 0.7071067811865476))
               ).astype(jnp.bfloat16)
        fc2 = jnp.dot(act, w2_ref[0], preferred_element_type=jnp.float32)
        fc2 = fc2 + b2_ref[0]
        val = fc2 * ew_ref[...][:, :1]
        out_ref[...] = jnp.where(mask, val, out_ref[...])


def _grouped_ffn(t_ids, g_ids, offs, nreal, xs, w1, b1, w2, b2, ews):
    grid_spec = pltpu.PrefetchScalarGridSpec(
        num_scalar_prefetch=4,
        grid=(NP,),
        in_specs=[
            pl.BlockSpec((TM, HIDDEN), lambda i, T_, G, O, N: (T_[i], 0)),
            pl.BlockSpec((1, HIDDEN, FFN), lambda i, T_, G, O, N: (G[i], 0, 0)),
            pl.BlockSpec((1, 1, FFN), lambda i, T_, G, O, N: (G[i], 0, 0)),
            pl.BlockSpec((1, FFN, HIDDEN), lambda i, T_, G, O, N: (G[i], 0, 0)),
            pl.BlockSpec((1, 1, HIDDEN), lambda i, T_, G, O, N: (G[i], 0, 0)),
            pl.BlockSpec((TM, 128), lambda i, T_, G, O, N: (T_[i], 0)),
        ],
        out_specs=pl.BlockSpec((TM, HIDDEN), lambda i, T_, G, O, N: (T_[i], 0)),
    )
    return pl.pallas_call(
        _moe_body,
        grid_spec=grid_spec,
        out_shape=jax.ShapeDtypeStruct((T, HIDDEN), jnp.float32),
        compiler_params=pltpu.CompilerParams(
            dimension_semantics=("arbitrary",)),
    )(t_ids, g_ids, offs, nreal, xs, w1, b1, w2, b2, ews)


def _pair_metadata(offs, counts):
    g_arange = jnp.arange(E, dtype=jnp.int32)
    nonempty = counts > 0
    s_g = jnp.where(nonempty, offs[:-1] // TM, 0)
    l_g = jnp.where(nonempty, (offs[1:] - 1) // TM, -1)
    span = jnp.where(nonempty, l_g - s_g + 1, 0)
    pair_start = jnp.concatenate(
        [jnp.zeros((1,), jnp.int32), jnp.cumsum(span)[:-1].astype(jnp.int32)])
    nreal = jnp.sum(span).astype(jnp.int32)
    idx = jnp.arange(NP, dtype=jnp.int32)
    gid = (jnp.searchsorted(pair_start, idx, side="right").astype(jnp.int32)
           - 1)
    gid = jnp.clip(gid, 0, E - 1)
    tid = s_g[gid] + (idx - pair_start[gid])
    valid = idx < nreal
    sort_key = jnp.where(valid, tid, NT)
    order = jnp.argsort(sort_key, stable=True)
    gid = gid[order]
    tid = jnp.where(valid[order], tid[order], 0)
    last_g = gid[jnp.maximum(nreal - 1, 0)]
    last_t = tid[jnp.maximum(nreal - 1, 0)]
    gid = jnp.where(idx < nreal, gid, last_g)
    tid = jnp.where(idx < nreal, tid, last_t)
    return tid, gid, nreal[None]


def kernel(hidden_states, expert_weights, w1, b1, w2, b2, top_experts):
    hidden_shape = hidden_states.shape
    hs = hidden_states.reshape(-1, HIDDEN)
    top = top_experts.reshape(-1).astype(jnp.int32)
    ew = expert_weights.reshape(-1)

    hist = _sc_histogram(top)
    counts = jnp.sum(hist, axis=0).astype(jnp.int32)
    offs = jnp.concatenate(
        [jnp.zeros((1,), jnp.int32), jnp.cumsum(counts).astype(jnp.int32)])
    start = (offs[:E][None, :]
             + jnp.cumsum(hist, axis=0).astype(jnp.int32) - hist)
    tid, gid, nreal = _pair_metadata(offs, counts)

    hs_bits = jax.lax.bitcast_convert_type(
        hs.astype(jnp.bfloat16).reshape(T, _XW, 2), jnp.int32)
    ew128 = jnp.broadcast_to(ew[:, None], (T, 128))
    xs_bits, ews128, inv2d = _sc_place_scatter(top, start, hs_bits, ew128)
    xs = jax.lax.bitcast_convert_type(
        xs_bits, jnp.bfloat16).reshape(T, HIDDEN)
    ews = ews128

    out_sorted = _grouped_ffn(tid, gid, offs, nreal, xs,
                              w1.astype(jnp.bfloat16), b1[:, None, :],
                              w2.astype(jnp.bfloat16), b2[:, None, :], ews)

    out = _make_row_gather(T, HIDDEN, jnp.float32, 32)(
        out_sorted, inv2d.reshape(T // 32, 32))
    return out.reshape(hidden_shape)
